# baseline (speedup 1.0000x reference)
import jax
import jax.numpy as jnp
from jax import lax
from jax.experimental import pallas as pl
from jax.experimental.pallas import tpu as pltpu

N = 54
D = 512
E = 864
H1 = 85
NODES_PER_STEP = 6
STEPS = N // NODES_PER_STEP

_PREC_LAYER = lax.Precision.HIGHEST
_PREC_EXACT = lax.Precision.DEFAULT


def _net_kernel(edge_ref, x_ref, g_ref,
                w1_ref, b1_ref, w2_ref, b2_ref, w3_ref, b3_ref,
                wg1_ref, bg1_ref, wg2_ref, bg2_ref, wg3_ref, bg3_ref,
                wo1_ref, wo1t_ref, bo1_ref, wo2_ref, bo2_ref,
                o_ref, emb_ref, acc_ref):
    j = pl.program_id(0)

    @pl.when(j == 0)
    def _prologue():
        src = edge_ref[0:1, :]
        dst = edge_ref[1:2, :]
        iota = lax.broadcasted_iota(jnp.int32, (N, E), 0)
        s_t = (iota == src).astype(jnp.float32)
        d_t = (iota == dst).astype(jnp.float32)
        a = lax.dot_general(d_t, s_t, (((1,), (1,)), ((), ())),
                            preferred_element_type=jnp.float32,
                            precision=_PREC_EXACT)
        deg_out = jnp.sum(s_t, axis=1, keepdims=True)
        deg_in = jnp.sum(d_t, axis=1, keepdims=True)
        r_out = lax.rsqrt(jnp.maximum(deg_out, 1.0))
        r_in = lax.rsqrt(jnp.maximum(deg_in, 1.0))

        h = x_ref[...]
        for w_ref, b_ref in ((w1_ref, b1_ref), (w2_ref, b2_ref),
                             (w3_ref, b3_ref)):
            hs = h * r_out
            agg = jnp.dot(a, hs, preferred_element_type=jnp.float32,
                          precision=_PREC_LAYER)
            agg = agg * r_in
            h = jnp.maximum(
                jnp.dot(agg, w_ref[...], preferred_element_type=jnp.float32,
                        precision=_PREC_LAYER) + b_ref[...], 0.0)
        emb_ref[...] = h

        g = g_ref[...]
        g = jnp.maximum(jnp.dot(g, wg1_ref[...], precision=_PREC_LAYER)
                        + bg1_ref[...], 0.0)
        g = jnp.maximum(jnp.dot(g, wg2_ref[...], precision=_PREC_LAYER)
                        + bg2_ref[...], 0.0)
        g = jnp.maximum(jnp.dot(g, wg3_ref[...], precision=_PREC_LAYER)
                        + bg3_ref[...], 0.0)
        acc_ref[...] = (jnp.dot(g, wo1t_ref[...], precision=_PREC_LAYER)
                        + bo1_ref[...])

    emb = emb_ref[...]
    row_iota = lax.broadcasted_iota(jnp.int32, (N, 1), 0)
    acc = acc_ref[...]
    for t in range(NODES_PER_STEP):
        n = j * NODES_PER_STEP + t
        mask = (row_iota == n).astype(jnp.float32)
        row = jnp.sum(emb * mask, axis=0, keepdims=True)
        blk = wo1_ref[t * D:(t + 1) * D, :]
        acc = acc + jnp.dot(row, blk, preferred_element_type=jnp.float32,
                            precision=_PREC_EXACT)
    acc_ref[...] = acc

    @pl.when(j == STEPS - 1)
    def _epilogue():
        out1 = jnp.maximum(acc_ref[...], 0.0)
        val = jnp.dot(out1, wo2_ref[...], precision=_PREC_LAYER) + bo2_ref[...]
        o_ref[...] = jax.nn.sigmoid(val)


def kernel(x, edge_index, global_feats, W1, b1, W2, b2, W3, b3,
           Wg1, bg1, Wg2, bg2, Wg3, bg3, Wo1, bo1, Wo2, bo2):
    const = lambda shape: pl.BlockSpec(shape, lambda j: (0,) * len(shape))
    blk_rows = NODES_PER_STEP * D
    out = pl.pallas_call(
        _net_kernel,
        grid=(STEPS,),
        in_specs=[
            const((2, E)), const((N, D)), const((1, 32)),
            const((D, D)), const((1, D)),
            const((D, D)), const((1, D)),
            const((D, D)), const((1, D)),
            const((32, 16)), const((1, 16)),
            const((16, 16)), const((1, 16)),
            const((16, 32)), const((1, 32)),
            pl.BlockSpec((blk_rows, H1), lambda j: (j, 0)),
            const((32, H1)), const((1, H1)),
            const((H1, 1)), const((1, 1)),
        ],
        out_specs=pl.BlockSpec((1, 1), lambda j: (0, 0)),
        out_shape=jax.ShapeDtypeStruct((1, 1), jnp.float32),
        scratch_shapes=[
            pltpu.VMEM((N, D), jnp.float32),
            pltpu.VMEM((1, H1), jnp.float32),
        ],
        compiler_params=pltpu.CompilerParams(
            dimension_semantics=("arbitrary",),
        ),
    )(edge_index, x, global_feats.reshape(1, -1),
      W1, b1.reshape(1, -1), W2, b2.reshape(1, -1), W3, b3.reshape(1, -1),
      Wg1, bg1.reshape(1, -1), Wg2, bg2.reshape(1, -1),
      Wg3, bg3.reshape(1, -1),
      Wo1, Wo1[N * D:], bo1.reshape(1, -1), Wo2,
      bo2.reshape(1, -1))
    return out.reshape(1)


# speedup vs baseline: 1.0999x; 1.0999x over previous
import jax
import jax.numpy as jnp
from jax import lax
from jax.experimental import pallas as pl
from jax.experimental.pallas import tpu as pltpu

N = 54
D = 512
E = 864
H1 = 85
NODES_PER_STEP = 18
STEPS = N // NODES_PER_STEP

_PREC_LAYER = lax.Precision.HIGHEST
_PREC_EXACT = lax.Precision.DEFAULT


def _net_kernel(edge_ref, x_ref, g_ref,
                w1_ref, b1_ref, w2_ref, b2_ref, w3_ref, b3_ref,
                wg1_ref, bg1_ref, wg2_ref, bg2_ref, wg3_ref, bg3_ref,
                wo1_ref, wo1t_ref, bo1_ref, wo2_ref, bo2_ref,
                o_ref, emb_ref, acc_ref):
    j = pl.program_id(0)

    @pl.when(j == 0)
    def _prologue():
        src = edge_ref[0:1, :]
        dst = edge_ref[1:2, :]
        iota = lax.broadcasted_iota(jnp.int32, (N, E), 0)
        s_t = (iota == src).astype(jnp.float32)
        d_t = (iota == dst).astype(jnp.float32)
        a = lax.dot_general(d_t, s_t, (((1,), (1,)), ((), ())),
                            preferred_element_type=jnp.float32,
                            precision=_PREC_EXACT)
        deg_out = jnp.sum(s_t, axis=1, keepdims=True)
        deg_in = jnp.sum(d_t, axis=1, keepdims=True)
        r_out = lax.rsqrt(jnp.maximum(deg_out, 1.0))
        r_in = lax.rsqrt(jnp.maximum(deg_in, 1.0))

        h = x_ref[...]
        for w_ref, b_ref in ((w1_ref, b1_ref), (w2_ref, b2_ref),
                             (w3_ref, b3_ref)):
            hs = h * r_out
            agg = jnp.dot(a, hs, preferred_element_type=jnp.float32,
                          precision=_PREC_LAYER)
            agg = agg * r_in
            h = jnp.maximum(
                jnp.dot(agg, w_ref[...], preferred_element_type=jnp.float32,
                        precision=_PREC_LAYER) + b_ref[...], 0.0)
        emb_ref[...] = h

        g = g_ref[...]
        g = jnp.maximum(jnp.dot(g, wg1_ref[...], precision=_PREC_LAYER)
                        + bg1_ref[...], 0.0)
        g = jnp.maximum(jnp.dot(g, wg2_ref[...], precision=_PREC_LAYER)
                        + bg2_ref[...], 0.0)
        g = jnp.maximum(jnp.dot(g, wg3_ref[...], precision=_PREC_LAYER)
                        + bg3_ref[...], 0.0)
        acc_ref[...] = (jnp.dot(g, wo1t_ref[...], precision=_PREC_LAYER)
                        + bo1_ref[...])

    emb = emb_ref[...]
    row_iota = lax.broadcasted_iota(jnp.int32, (N, 1), 0)
    acc = acc_ref[...]
    for t in range(NODES_PER_STEP):
        n = j * NODES_PER_STEP + t
        mask = (row_iota == n).astype(jnp.float32)
        row = jnp.sum(emb * mask, axis=0, keepdims=True)
        blk = wo1_ref[t * D:(t + 1) * D, :]
        acc = acc + jnp.dot(row, blk, preferred_element_type=jnp.float32,
                            precision=_PREC_EXACT)
    acc_ref[...] = acc

    @pl.when(j == STEPS - 1)
    def _epilogue():
        out1 = jnp.maximum(acc_ref[...], 0.0)
        val = jnp.dot(out1, wo2_ref[...], precision=_PREC_LAYER) + bo2_ref[...]
        o_ref[...] = jax.nn.sigmoid(val)


def kernel(x, edge_index, global_feats, W1, b1, W2, b2, W3, b3,
           Wg1, bg1, Wg2, bg2, Wg3, bg3, Wo1, bo1, Wo2, bo2):
    const = lambda shape: pl.BlockSpec(shape, lambda j: (0,) * len(shape))
    blk_rows = NODES_PER_STEP * D
    out = pl.pallas_call(
        _net_kernel,
        grid=(STEPS,),
        in_specs=[
            const((2, E)), const((N, D)), const((1, 32)),
            const((D, D)), const((1, D)),
            const((D, D)), const((1, D)),
            const((D, D)), const((1, D)),
            const((32, 16)), const((1, 16)),
            const((16, 16)), const((1, 16)),
            const((16, 32)), const((1, 32)),
            pl.BlockSpec((blk_rows, H1), lambda j: (j, 0)),
            const((32, H1)), const((1, H1)),
            const((H1, 1)), const((1, 1)),
        ],
        out_specs=pl.BlockSpec((1, 1), lambda j: (0, 0)),
        out_shape=jax.ShapeDtypeStruct((1, 1), jnp.float32),
        scratch_shapes=[
            pltpu.VMEM((N, D), jnp.float32),
            pltpu.VMEM((1, H1), jnp.float32),
        ],
        compiler_params=pltpu.CompilerParams(
            dimension_semantics=("arbitrary",),
        ),
    )(edge_index, x, global_feats.reshape(1, -1),
      W1, b1.reshape(1, -1), W2, b2.reshape(1, -1), W3, b3.reshape(1, -1),
      Wg1, bg1.reshape(1, -1), Wg2, bg2.reshape(1, -1),
      Wg3, bg3.reshape(1, -1),
      Wo1, Wo1[N * D:], bo1.reshape(1, -1), Wo2,
      bo2.reshape(1, -1))
    return out.reshape(1)


# speedup vs baseline: 1.1516x; 1.0470x over previous
import jax
import jax.numpy as jnp
from jax import lax
from jax.experimental import pallas as pl
from jax.experimental.pallas import tpu as pltpu

N = 54
D = 512
E = 864
H1 = 85
NODES_PER_STEP = 27
STEPS = N // NODES_PER_STEP

_PREC_LAYER = lax.Precision.HIGHEST
_PREC_EXACT = lax.Precision.DEFAULT


def _net_kernel(edge_ref, x_ref, g_ref,
                w1_ref, b1_ref, w2_ref, b2_ref, w3_ref, b3_ref,
                wg1_ref, bg1_ref, wg2_ref, bg2_ref, wg3_ref, bg3_ref,
                wo1_ref, wo1t_ref, bo1_ref, wo2_ref, bo2_ref,
                o_ref, emb_ref, acc_ref):
    j = pl.program_id(0)

    @pl.when(j == 0)
    def _prologue():
        src = edge_ref[0:1, :]
        dst = edge_ref[1:2, :]
        iota = lax.broadcasted_iota(jnp.int32, (N, E), 0)
        s_t = (iota == src).astype(jnp.float32)
        d_t = (iota == dst).astype(jnp.float32)
        a = lax.dot_general(d_t, s_t, (((1,), (1,)), ((), ())),
                            preferred_element_type=jnp.float32,
                            precision=_PREC_EXACT)
        deg_out = jnp.sum(s_t, axis=1, keepdims=True)
        deg_in = jnp.sum(d_t, axis=1, keepdims=True)
        r_out = lax.rsqrt(jnp.maximum(deg_out, 1.0))
        r_in = lax.rsqrt(jnp.maximum(deg_in, 1.0))

        h = x_ref[...]
        for w_ref, b_ref in ((w1_ref, b1_ref), (w2_ref, b2_ref),
                             (w3_ref, b3_ref)):
            hs = h * r_out
            agg = jnp.dot(a, hs, preferred_element_type=jnp.float32,
                          precision=_PREC_LAYER)
            agg = agg * r_in
            h = jnp.maximum(
                jnp.dot(agg, w_ref[...], preferred_element_type=jnp.float32,
                        precision=_PREC_LAYER) + b_ref[...], 0.0)
        emb_ref[...] = h

        g = g_ref[...]
        g = jnp.maximum(jnp.dot(g, wg1_ref[...], precision=_PREC_LAYER)
                        + bg1_ref[...], 0.0)
        g = jnp.maximum(jnp.dot(g, wg2_ref[...], precision=_PREC_LAYER)
                        + bg2_ref[...], 0.0)
        g = jnp.maximum(jnp.dot(g, wg3_ref[...], precision=_PREC_LAYER)
                        + bg3_ref[...], 0.0)
        acc_ref[...] = (jnp.dot(g, wo1t_ref[...], precision=_PREC_LAYER)
                        + bo1_ref[...])

    emb = emb_ref[...]
    row_iota = lax.broadcasted_iota(jnp.int32, (N, 1), 0)
    acc = acc_ref[...]
    for t in range(NODES_PER_STEP):
        n = j * NODES_PER_STEP + t
        mask = (row_iota == n).astype(jnp.float32)
        row = jnp.sum(emb * mask, axis=0, keepdims=True)
        blk = wo1_ref[t * D:(t + 1) * D, :]
        acc = acc + jnp.dot(row, blk, preferred_element_type=jnp.float32,
                            precision=_PREC_EXACT)
    acc_ref[...] = acc

    @pl.when(j == STEPS - 1)
    def _epilogue():
        out1 = jnp.maximum(acc_ref[...], 0.0)
        val = jnp.dot(out1, wo2_ref[...], precision=_PREC_LAYER) + bo2_ref[...]
        o_ref[...] = jax.nn.sigmoid(val)


def kernel(x, edge_index, global_feats, W1, b1, W2, b2, W3, b3,
           Wg1, bg1, Wg2, bg2, Wg3, bg3, Wo1, bo1, Wo2, bo2):
    const = lambda shape: pl.BlockSpec(shape, lambda j: (0,) * len(shape))
    blk_rows = NODES_PER_STEP * D
    out = pl.pallas_call(
        _net_kernel,
        grid=(STEPS,),
        in_specs=[
            const((2, E)), const((N, D)), const((1, 32)),
            const((D, D)), const((1, D)),
            const((D, D)), const((1, D)),
            const((D, D)), const((1, D)),
            const((32, 16)), const((1, 16)),
            const((16, 16)), const((1, 16)),
            const((16, 32)), const((1, 32)),
            pl.BlockSpec((blk_rows, H1), lambda j: (j, 0)),
            const((32, H1)), const((1, H1)),
            const((H1, 1)), const((1, 1)),
        ],
        out_specs=pl.BlockSpec((1, 1), lambda j: (0, 0)),
        out_shape=jax.ShapeDtypeStruct((1, 1), jnp.float32),
        scratch_shapes=[
            pltpu.VMEM((N, D), jnp.float32),
            pltpu.VMEM((1, H1), jnp.float32),
        ],
        compiler_params=pltpu.CompilerParams(
            dimension_semantics=("arbitrary",),
        ),
    )(edge_index, x, global_feats.reshape(1, -1),
      W1, b1.reshape(1, -1), W2, b2.reshape(1, -1), W3, b3.reshape(1, -1),
      Wg1, bg1.reshape(1, -1), Wg2, bg2.reshape(1, -1),
      Wg3, bg3.reshape(1, -1),
      Wo1, Wo1[N * D:], bo1.reshape(1, -1), Wo2,
      bo2.reshape(1, -1))
    return out.reshape(1)
